# baseline (speedup 1.0000x reference)
import functools

import jax
import jax.numpy as jnp
from jax.experimental import pallas as pl
from jax.experimental.pallas import tpu as pltpu

_SPARSITY = 0.8
_THRESHOLD = 0.85
_BISECT_ITERS = 20
_TILE = 1024
_INV_SQRT2 = 0.7071067811865476


def _gelu(v):
    return 0.5 * v * (1.0 + jax.lax.erf(v * _INV_SQRT2))


def _dot(a, b):
    return jax.lax.dot_general(
        a, b, (((0,), (0,)), ((), ())),
        preferred_element_type=jnp.float32,
        precision=jax.lax.Precision.DEFAULT)


def _dot_std(a, b):
    return jax.lax.dot_general(
        a, b, (((1,), (0,)), ((), ())),
        preferred_element_type=jnp.float32,
        precision=jax.lax.Precision.DEFAULT)


def _kwta_mask_t(ht, k):
    ah = jnp.abs(ht)
    mx = jnp.max(ah, axis=0, keepdims=True)
    lo = jnp.zeros_like(mx)
    hi = mx * (1.0 + 2.0 ** -12) + 1e-30
    kf = jnp.float32(k)
    for _ in range(_BISECT_ITERS):
        mid = 0.5 * (lo + hi)
        cnt = jnp.sum((ah >= mid).astype(jnp.float32), axis=0, keepdims=True)
        pred = cnt >= kf
        lo = jnp.where(pred, mid, lo)
        hi = jnp.where(pred, hi, mid)
    return ah >= lo


def _conf_t(logits_t):
    m = jnp.max(logits_t, axis=0, keepdims=True)
    se = jnp.sum(jnp.exp(logits_t - m), axis=0, keepdims=True)
    return 1.0 / se


def _vlm_sq_t(xt, encWT, encbT, decWT, decbT):
    comp = jax.nn.relu(_dot_std(encWT, xt) + encbT)
    mim = _dot(decWT, comp) + decbT
    return jnp.sum((mim - xt) ** 2)


def _dca_t(xt, wt, bt, active, k):
    ht = _dot(wt, xt) + bt
    proc = xt + _gelu(ht * _kwta_mask_t(ht, k).astype(jnp.float32))
    return jnp.where(active > 0.0, proc, xt)


def _phase_a_kernel(x_ref, dcaWT_ref, dcabT_ref, cenWT_ref, cenbT_ref,
                    cohwT_ref, cohb_ref, eeWT_ref, eebT_ref, encWT_ref,
                    encbT_ref, decWT_ref, decbT_ref, x2_ref, proc3_ref,
                    flt_ref, act_ref, stats_ref, *, k, half):
    xt = x_ref[...].T
    tt = xt.shape[1]
    active = jnp.ones((1, tt), jnp.float32)
    encWT = encWT_ref[...]
    encbT = encbT_ref[...]
    decWT = decWT_ref[...]
    decbT = decbT_ref[...]

    ht = _dot(dcaWT_ref[0], xt) + dcabT_ref[0]
    gated = _gelu(ht * _kwta_mask_t(ht, k).astype(jnp.float32))
    for i in range(half):
        stats_ref[0, 0, 4 + i] = jnp.sum(active)
        xt = jnp.where(active > 0.0, xt + gated, xt)
        stats_ref[0, 0, 7 + i] = _vlm_sq_t(xt, encWT, encbT, decWT, decbT)
        ht = _dot(dcaWT_ref[i + 1], xt) + dcabT_ref[i + 1]
        gated = _gelu(ht * _kwta_mask_t(ht, k).astype(jnp.float32))
        logits_t = _dot_std(eeWT_ref[i], xt) + eebT_ref[i]
        conf = _conf_t(logits_t)
        if i == 0:
            flt_ref[...] = logits_t
        else:
            flt_ref[...] = jnp.where(active > 0.0, logits_t, flt_ref[...])
        active = active * (conf < _THRESHOLD).astype(jnp.float32)

    stats_ref[0, 0, 3] = jnp.sum(active)
    proc3 = xt + gated
    cohwT = cohwT_ref[...]
    cohb = cohb_ref[0, 0]
    for j in range(cenWT_ref.shape[0]):
        sims = _gelu(_dot(cenWT_ref[j], proc3) + cenbT_ref[j])
        coh = jnp.sum(sims * cohwT, axis=0, keepdims=True) + cohb
        stats_ref[0, 0, j] = jnp.sum(coh * active)

    x2_ref[...] = xt
    proc3_ref[...] = proc3
    act_ref[...] = active.reshape(1, 1, tt)


def _phase_b_kernel(astats_ref, x2_ref, proc3_ref, act_ref, flt_hbm, dcaWT_ref,
                    dcabT_ref, cenWT_ref, cenbT_ref, eeWT_ref, eebT_ref,
                    encWT_ref, encbT_ref, decWT_ref, decbT_ref, fl_ref,
                    stats_ref, flt_scr, dma_sem, *, k, n_layers, half, nb):
    x2 = x2_ref[...]
    proc3 = proc3_ref[...]
    tt = x2.shape[1]
    active = act_ref[0]
    encWT = encWT_ref[...]
    encbT = encbT_ref[...]
    decWT = decWT_ref[...]
    decbT = decbT_ref[...]

    ntiles = astats_ref.shape[0]
    def _score(j):
        def body(t, acc):
            return acc + astats_ref[t, 0, j]
        return jax.lax.fori_loop(0, ntiles, body, jnp.float32(0.0))
    best = jnp.int32(0)
    sbest = _score(0)
    for j in range(1, nb):
        sj = _score(j)
        best = jnp.where(sj > sbest, jnp.int32(j), best)
        sbest = jnp.maximum(sj, sbest)

    tile = pl.program_id(0)
    copy = pltpu.make_async_copy(
        flt_hbm.at[:, pl.ds(tile * tt, tt)], flt_scr, dma_sem)
    any_exited = jnp.min(active) < 0.5
    @pl.when(any_exited)
    def _():
        copy.start()
    @pl.when(any_exited)
    def _():
        copy.wait()

    sims = _gelu(_dot(cenWT_ref[best], proc3) + cenbT_ref[best])
    xt = jnp.where(active > 0.0, sims + proc3, x2)
    stats_ref[0, 0, 0] = _vlm_sq_t(xt, encWT, encbT, decWT, decbT)
    ht = _dot(dcaWT_ref[0], xt) + dcabT_ref[0]
    gated = _gelu(ht * _kwta_mask_t(ht, k).astype(jnp.float32))
    logits_t = _dot_std(eeWT_ref[0], xt) + eebT_ref[0]
    conf = _conf_t(logits_t)
    flt = jnp.where(active > 0.0, logits_t, flt_scr[...])
    active = active * (conf < _THRESHOLD).astype(jnp.float32)

    for i in range(half + 1, n_layers):
        li = i - half - 1
        stats_ref[0, 0, 3 + li] = jnp.sum(active)
        xt = jnp.where(active > 0.0, xt + gated, xt)
        stats_ref[0, 0, 1 + li] = _vlm_sq_t(xt, encWT, encbT, decWT, decbT)
        if i + 1 < n_layers:
            ht = _dot(dcaWT_ref[li + 1], xt) + dcabT_ref[li + 1]
            gated = _gelu(ht * _kwta_mask_t(ht, k).astype(jnp.float32))
        logits_t = _dot_std(eeWT_ref[i - half], xt) + eebT_ref[i - half]
        conf = _conf_t(logits_t)
        flt = jnp.where(active > 0.0, logits_t, flt)
        active = active * (conf < _THRESHOLD).astype(jnp.float32)

    fl_ref[0] = flt


def _const_spec(shape):
    nd = len(shape)
    return pl.BlockSpec(shape, lambda t: (0,) * nd)


def kernel(x, dca_W, dca_b, cen_W, cen_b, coh_w, coh_b, ee_W, ee_b,
           vlm_enc_W, vlm_enc_b, vlm_dec_W, vlm_dec_b):
    b, s, d = x.shape
    n_layers = dca_W.shape[0]
    half = n_layers // 2
    n_classes = ee_W.shape[-1]
    n = b * s
    k = max(1, int(d * (1.0 - _SPARSITY)))
    tt = _TILE
    g = n // tt

    xf = x.reshape(n, d)
    dcaWT = dca_W
    dcabT = dca_b[..., None]
    cenWT = cen_W
    cenbT = cen_b[..., None]
    eeWT = jnp.swapaxes(ee_W, 1, 2)
    eebT = ee_b[..., None]
    encWT = vlm_enc_W.T
    encbT = vlm_enc_b[:, None]
    decWT = vlm_dec_W
    decbT = vlm_dec_b[:, None]
    cohwT = coh_w[:, None]
    cohb2 = coh_b.reshape(1, 1)
    student = vlm_enc_W.shape[-1]
    nb = cen_W.shape[0]

    tcol = lambda t: (0, t)
    cparams = pltpu.CompilerParams(
        dimension_semantics=("arbitrary",),
        vmem_limit_bytes=56 * 1024 * 1024,
    )

    x2, proc3, flt_a, act, stats_a = pl.pallas_call(
        functools.partial(_phase_a_kernel, k=k, half=half),
        grid=(g,),
        in_specs=[
            pl.BlockSpec((tt, d), lambda t: (t, 0)),
            _const_spec((half + 1, d, d)),
            _const_spec((half + 1, d, 1)),
            _const_spec((nb, d, d)),
            _const_spec((nb, d, 1)),
            _const_spec((d, 1)),
            pl.BlockSpec(memory_space=pltpu.SMEM),
            _const_spec((half, n_classes, d)),
            _const_spec((half, n_classes, 1)),
            _const_spec((student, d)),
            _const_spec((student, 1)),
            _const_spec((student, d)),
            _const_spec((d, 1)),
        ],
        out_specs=[
            pl.BlockSpec((d, tt), tcol),
            pl.BlockSpec((d, tt), tcol),
            pl.BlockSpec((n_classes, tt), tcol),
            pl.BlockSpec((1, 1, tt), lambda t: (0, 0, t)),
            pl.BlockSpec((1, 1, 16), lambda t: (t, 0, 0),
                         memory_space=pltpu.SMEM),
        ],
        out_shape=[
            jax.ShapeDtypeStruct((d, n), jnp.float32),
            jax.ShapeDtypeStruct((d, n), jnp.float32),
            jax.ShapeDtypeStruct((n_classes, n), jnp.float32),
            jax.ShapeDtypeStruct((1, 1, n), jnp.float32),
            jax.ShapeDtypeStruct((g, 1, 16), jnp.float32),
        ],
        compiler_params=cparams,
    )(xf, dcaWT[:half + 1], dcabT[:half + 1], cenWT, cenbT, cohwT, cohb2,
      eeWT[:half], eebT[:half], encWT, encbT, decWT, decbT)

    fl, stats_b = pl.pallas_call(
        functools.partial(_phase_b_kernel, k=k, n_layers=n_layers, half=half,
                          nb=nb),
        grid=(g,),
        in_specs=[
            pl.BlockSpec(memory_space=pltpu.SMEM),
            pl.BlockSpec((d, tt), tcol),
            pl.BlockSpec((d, tt), tcol),
            pl.BlockSpec((1, 1, tt), lambda t: (0, 0, t)),
            pl.BlockSpec(memory_space=pl.ANY),
            _const_spec((n_layers - half - 1, d, d)),
            _const_spec((n_layers - half - 1, d, 1)),
            _const_spec((nb, d, d)),
            _const_spec((nb, d, 1)),
            _const_spec((n_layers - half, n_classes, d)),
            _const_spec((n_layers - half, n_classes, 1)),
            _const_spec((student, d)),
            _const_spec((student, 1)),
            _const_spec((student, d)),
            _const_spec((d, 1)),
        ],
        out_specs=[
            pl.BlockSpec((1, n_classes, tt),
                         lambda t, _spt=s // tt: (t // _spt, 0, t % _spt)),
            pl.BlockSpec((1, 1, 16), lambda t: (t, 0, 0),
                         memory_space=pltpu.SMEM),
        ],
        out_shape=[
            jax.ShapeDtypeStruct((b, n_classes, s), jnp.float32),
            jax.ShapeDtypeStruct((g, 1, 16), jnp.float32),
        ],
        compiler_params=cparams,
        scratch_shapes=[pltpu.VMEM((n_classes, tt), jnp.float32),
                        pltpu.SemaphoreType.DMA],
    )(stats_a, x2, proc3, act, flt_a, dcaWT[half + 1:], dcabT[half + 1:],
      cenWT, cenbT, eeWT[half:], eebT[half:], encWT, encbT, decWT, decbT)

    nact = jnp.stack([jnp.sum(stats_a[:, 0, 4]), jnp.sum(stats_a[:, 0, 5]),
                      jnp.sum(stats_a[:, 0, 6]), jnp.sum(stats_a[:, 0, 3]),
                      jnp.sum(stats_b[:, 0, 3]), jnp.sum(stats_b[:, 0, 4])])
    sq = jnp.stack([jnp.sum(stats_a[:, 0, 7]), jnp.sum(stats_a[:, 0, 8]),
                    jnp.sum(stats_a[:, 0, 9]), jnp.sum(stats_b[:, 0, 0]),
                    jnp.sum(stats_b[:, 0, 1]), jnp.sum(stats_b[:, 0, 2])])
    any_act = (nact > 0.0).astype(jnp.float32)
    vloss = sq / jnp.float32(n * d)
    loss_sum = jnp.sum(vloss * any_act)
    cnt = jnp.sum(any_act)
    avg_layers = jnp.sum(nact) / jnp.float32(n)
    avg_vloss = loss_sum / jnp.maximum(cnt, 1.0)
    return jnp.transpose(fl, (0, 2, 1)), avg_layers, avg_vloss


# speedup vs baseline: 1.0325x; 1.0325x over previous
import functools

import jax
import jax.numpy as jnp
from jax.experimental import pallas as pl
from jax.experimental.pallas import tpu as pltpu

_SPARSITY = 0.8
_THRESHOLD = 0.85
_BISECT_ITERS = 20
_TILE = 1024
_INV_SQRT2 = 0.7071067811865476


def _gelu(v):
    return 0.5 * v * (1.0 + jax.lax.erf(v * _INV_SQRT2))


def _dot(a, b):
    return jax.lax.dot_general(
        a, b, (((0,), (0,)), ((), ())),
        preferred_element_type=jnp.float32,
        precision=jax.lax.Precision.DEFAULT)


def _dot_std(a, b):
    return jax.lax.dot_general(
        a, b, (((1,), (0,)), ((), ())),
        preferred_element_type=jnp.float32,
        precision=jax.lax.Precision.DEFAULT)


def _kwta_mask_t(ht, k):
    ah = jnp.abs(ht)
    mx = jnp.max(ah, axis=0, keepdims=True)
    lo = jnp.zeros_like(mx)
    hi = mx * (1.0 + 2.0 ** -12) + 1e-30
    kf = jnp.float32(k)
    for _ in range(_BISECT_ITERS):
        mid = 0.5 * (lo + hi)
        cnt = jnp.sum((ah >= mid).astype(jnp.float32), axis=0, keepdims=True)
        pred = cnt >= kf
        lo = jnp.where(pred, mid, lo)
        hi = jnp.where(pred, hi, mid)
    return ah >= lo


def _conf_t(logits_t):
    m = jnp.max(logits_t, axis=0, keepdims=True)
    se = jnp.sum(jnp.exp(logits_t - m), axis=0, keepdims=True)
    return 1.0 / se


def _vlm_sq_t(xt, encWT, encbT, decWT, decbT):
    comp = jax.nn.relu(_dot_std(encWT, xt) + encbT)
    mim = _dot(decWT, comp) + decbT
    return jnp.sum((mim - xt) ** 2)


def _dca_t(xt, wt, bt, active, k):
    ht = _dot(wt, xt) + bt
    proc = xt + _gelu(ht * _kwta_mask_t(ht, k).astype(jnp.float32))
    return jnp.where(active > 0.0, proc, xt)


def _phase_a_kernel(x_ref, dcaWT_ref, dcabT_ref, cenWT_ref, cenbT_ref,
                    cohwT_ref, cohb_ref, eeWT_ref, eebT_ref, encWT_ref,
                    encbT_ref, decWT_ref, decbT_ref, x2_ref, proc3_ref,
                    flt_ref, act_ref, stats_ref, *, k, half):
    xt = x_ref[...].T
    tt = xt.shape[1]
    active = jnp.ones((1, tt), jnp.float32)
    encWT = encWT_ref[...]
    encbT = encbT_ref[...]
    decWT = decWT_ref[...]
    decbT = decbT_ref[...]

    for i in range(half):
        stats_ref[0, 0, 4 + i] = jnp.sum(active)
        xt = _dca_t(xt, dcaWT_ref[i], dcabT_ref[i], active, k)
        stats_ref[0, 0, 7 + i] = _vlm_sq_t(xt, encWT, encbT, decWT, decbT)
        logits_t = _dot_std(eeWT_ref[i], xt) + eebT_ref[i]
        conf = _conf_t(logits_t)
        if i == 0:
            flt_ref[...] = logits_t
        else:
            flt_ref[...] = jnp.where(active > 0.0, logits_t, flt_ref[...])
        active = active * (conf < _THRESHOLD).astype(jnp.float32)

    stats_ref[0, 0, 3] = jnp.sum(active)
    ht = _dot(dcaWT_ref[half], xt) + dcabT_ref[half]
    proc3 = xt + _gelu(ht * _kwta_mask_t(ht, k).astype(jnp.float32))
    cohwT = cohwT_ref[...]
    cohb = cohb_ref[0, 0]
    for j in range(cenWT_ref.shape[0]):
        sims = _gelu(_dot(cenWT_ref[j], proc3) + cenbT_ref[j])
        coh = jnp.sum(sims * cohwT, axis=0, keepdims=True) + cohb
        stats_ref[0, 0, j] = jnp.sum(coh * active)

    x2_ref[...] = xt
    proc3_ref[...] = proc3
    act_ref[...] = active.reshape(1, 1, tt)


def _phase_b_kernel(best_ref, x2_ref, proc3_ref, act_ref, flt_ref, dcaWT_ref,
                    dcabT_ref, cenWT_ref, cenbT_ref, eeWT_ref, eebT_ref,
                    encWT_ref, encbT_ref, decWT_ref, decbT_ref, fl_ref,
                    stats_ref, *, k, n_layers, half):
    x2 = x2_ref[...]
    proc3 = proc3_ref[...]
    active = act_ref[0]
    encWT = encWT_ref[...]
    encbT = encbT_ref[...]
    decWT = decWT_ref[...]
    decbT = decbT_ref[...]
    best = best_ref[0]

    sims = _gelu(_dot(cenWT_ref[best], proc3) + cenbT_ref[best])
    xt = jnp.where(active > 0.0, sims + proc3, x2)
    stats_ref[0, 0, 0] = _vlm_sq_t(xt, encWT, encbT, decWT, decbT)
    logits_t = _dot_std(eeWT_ref[0], xt) + eebT_ref[0]
    conf = _conf_t(logits_t)
    flt = jnp.where(active > 0.0, logits_t, flt_ref[...])
    active = active * (conf < _THRESHOLD).astype(jnp.float32)

    for i in range(half + 1, n_layers):
        li = i - half - 1
        stats_ref[0, 0, 3 + li] = jnp.sum(active)
        xt = _dca_t(xt, dcaWT_ref[li], dcabT_ref[li], active, k)
        stats_ref[0, 0, 1 + li] = _vlm_sq_t(xt, encWT, encbT, decWT, decbT)
        logits_t = _dot_std(eeWT_ref[i - half], xt) + eebT_ref[i - half]
        conf = _conf_t(logits_t)
        flt = jnp.where(active > 0.0, logits_t, flt)
        active = active * (conf < _THRESHOLD).astype(jnp.float32)

    fl_ref[0] = flt


def _const_spec(shape):
    nd = len(shape)
    return pl.BlockSpec(shape, lambda t: (0,) * nd)


def kernel(x, dca_W, dca_b, cen_W, cen_b, coh_w, coh_b, ee_W, ee_b,
           vlm_enc_W, vlm_enc_b, vlm_dec_W, vlm_dec_b):
    b, s, d = x.shape
    n_layers = dca_W.shape[0]
    half = n_layers // 2
    n_classes = ee_W.shape[-1]
    n = b * s
    k = max(1, int(d * (1.0 - _SPARSITY)))
    tt = _TILE
    g = n // tt

    xf = x.reshape(n, d)
    dcaWT = dca_W
    dcabT = dca_b[..., None]
    cenWT = cen_W
    cenbT = cen_b[..., None]
    eeWT = jnp.swapaxes(ee_W, 1, 2)
    eebT = ee_b[..., None]
    encWT = vlm_enc_W.T
    encbT = vlm_enc_b[:, None]
    decWT = vlm_dec_W
    decbT = vlm_dec_b[:, None]
    cohwT = coh_w[:, None]
    cohb2 = coh_b.reshape(1, 1)
    student = vlm_enc_W.shape[-1]
    nb = cen_W.shape[0]

    tcol = lambda t: (0, t)
    cparams = pltpu.CompilerParams(
        dimension_semantics=("arbitrary",),
        vmem_limit_bytes=56 * 1024 * 1024,
    )

    x2, proc3, flt_a, act, stats_a = pl.pallas_call(
        functools.partial(_phase_a_kernel, k=k, half=half),
        grid=(g,),
        in_specs=[
            pl.BlockSpec((tt, d), lambda t: (t, 0)),
            _const_spec((half + 1, d, d)),
            _const_spec((half + 1, d, 1)),
            _const_spec((nb, d, d)),
            _const_spec((nb, d, 1)),
            _const_spec((d, 1)),
            pl.BlockSpec(memory_space=pltpu.SMEM),
            _const_spec((half, n_classes, d)),
            _const_spec((half, n_classes, 1)),
            _const_spec((student, d)),
            _const_spec((student, 1)),
            _const_spec((student, d)),
            _const_spec((d, 1)),
        ],
        out_specs=[
            pl.BlockSpec((d, tt), tcol),
            pl.BlockSpec((d, tt), tcol),
            pl.BlockSpec((n_classes, tt), tcol),
            pl.BlockSpec((1, 1, tt), lambda t: (0, 0, t)),
            pl.BlockSpec((1, 1, 16), lambda t: (t, 0, 0),
                         memory_space=pltpu.SMEM),
        ],
        out_shape=[
            jax.ShapeDtypeStruct((d, n), jnp.float32),
            jax.ShapeDtypeStruct((d, n), jnp.float32),
            jax.ShapeDtypeStruct((n_classes, n), jnp.float32),
            jax.ShapeDtypeStruct((1, 1, n), jnp.float32),
            jax.ShapeDtypeStruct((g, 1, 16), jnp.float32),
        ],
        compiler_params=cparams,
    )(xf, dcaWT[:half + 1], dcabT[:half + 1], cenWT, cenbT, cohwT, cohb2,
      eeWT[:half], eebT[:half], encWT, encbT, decWT, decbT)

    nact3 = jnp.sum(stats_a[:, 0, 3])
    denom = jnp.maximum(nact3, 1.0)
    scores = jnp.sum(stats_a[:, 0, :nb], axis=0) / denom
    best = jnp.argmax(scores).astype(jnp.int32).reshape(1)

    fl, stats_b = pl.pallas_call(
        functools.partial(_phase_b_kernel, k=k, n_layers=n_layers, half=half),
        grid=(g,),
        in_specs=[
            pl.BlockSpec(memory_space=pltpu.SMEM),
            pl.BlockSpec((d, tt), tcol),
            pl.BlockSpec((d, tt), tcol),
            pl.BlockSpec((1, 1, tt), lambda t: (0, 0, t)),
            pl.BlockSpec((n_classes, tt), tcol),
            _const_spec((n_layers - half - 1, d, d)),
            _const_spec((n_layers - half - 1, d, 1)),
            _const_spec((nb, d, d)),
            _const_spec((nb, d, 1)),
            _const_spec((n_layers - half, n_classes, d)),
            _const_spec((n_layers - half, n_classes, 1)),
            _const_spec((student, d)),
            _const_spec((student, 1)),
            _const_spec((student, d)),
            _const_spec((d, 1)),
        ],
        out_specs=[
            pl.BlockSpec((1, n_classes, tt),
                         lambda t, _spt=s // tt: (t // _spt, 0, t % _spt)),
            pl.BlockSpec((1, 1, 16), lambda t: (t, 0, 0),
                         memory_space=pltpu.SMEM),
        ],
        out_shape=[
            jax.ShapeDtypeStruct((b, n_classes, s), jnp.float32),
            jax.ShapeDtypeStruct((g, 1, 16), jnp.float32),
        ],
        compiler_params=cparams,
    )(best, x2, proc3, act, flt_a, dcaWT[half + 1:], dcabT[half + 1:],
      cenWT, cenbT, eeWT[half:], eebT[half:], encWT, encbT, decWT, decbT)

    nact = jnp.stack([jnp.sum(stats_a[:, 0, 4]), jnp.sum(stats_a[:, 0, 5]),
                      jnp.sum(stats_a[:, 0, 6]), nact3,
                      jnp.sum(stats_b[:, 0, 3]), jnp.sum(stats_b[:, 0, 4])])
    sq = jnp.stack([jnp.sum(stats_a[:, 0, 7]), jnp.sum(stats_a[:, 0, 8]),
                    jnp.sum(stats_a[:, 0, 9]), jnp.sum(stats_b[:, 0, 0]),
                    jnp.sum(stats_b[:, 0, 1]), jnp.sum(stats_b[:, 0, 2])])
    any_act = (nact > 0.0).astype(jnp.float32)
    vloss = sq / jnp.float32(n * d)
    loss_sum = jnp.sum(vloss * any_act)
    cnt = jnp.sum(any_act)
    avg_layers = jnp.sum(nact) / jnp.float32(n)
    avg_vloss = loss_sum / jnp.maximum(cnt, 1.0)
    return jnp.transpose(fl, (0, 2, 1)), avg_layers, avg_vloss
